# baseline (speedup 1.0000x reference)
import functools
import math

import jax
import jax.numpy as jnp
from jax import lax
from jax.experimental import pallas as pl
from jax.experimental.pallas import tpu as pltpu
from jax.experimental.pallas import tpu_sc as plsc

D_MODEL = 64
SCALE = math.sqrt(D_MODEL)

G = 128
K = 2
C = G * K


@functools.lru_cache(maxsize=None)
def _build(B):
    info = plsc.get_sparse_core_info()
    NW = info.num_cores * info.num_subcores
    assert B % (NW * C * 2) == 0
    b_per_w = B // NW
    n_chunks = b_per_w // C
    n_groups = n_chunks // 2

    mesh = plsc.VectorSubcoreMesh(core_axis_name="c", subcore_axis_name="s")

    @functools.partial(
        pl.kernel,
        mesh=mesh,
        compiler_params=pltpu.CompilerParams(use_tc_tiling_on_sc=False),
        out_type=jax.ShapeDtypeStruct((B // 2, 2 * D_MODEL), jnp.float32),
        scratch_types=[
            pltpu.VMEM((b_per_w,), jnp.int32),
            pltpu.VMEM((2, C, D_MODEL), jnp.float32),
            pltpu.VMEM((2, C // 2, 2 * D_MODEL), jnp.float32),
            pltpu.SemaphoreType.DMA,
            pltpu.SemaphoreType.DMA,
            pltpu.SemaphoreType.DMA,
            pltpu.SemaphoreType.DMA,
        ],
    )
    def emb_kernel(idx_hbm, table_hbm, out_hbm, idx_v, ga_v, sc_v, *sems):
        gsem = sems[:2]
        osem = sems[2:]
        cid = lax.axis_index("c")
        sid = lax.axis_index("s")
        wid = sid * info.num_cores + cid
        row_base = wid * b_per_w

        pltpu.sync_copy(idx_hbm.at[pl.ds(row_base, b_per_w)], idx_v)

        def fire_gather(ci, s):
            for j in range(K):
                pltpu.async_copy(
                    table_hbm.at[idx_v.at[pl.ds((ci * K + j) * G, G)]],
                    ga_v.at[s, pl.ds(j * G, G)],
                    gsem[s],
                )

        def wait_gather(s):
            pltpu.make_async_copy(
                table_hbm.at[pl.ds(0, C)], ga_v.at[s], gsem[s]).wait()

        def out_slice(ci):
            return out_hbm.at[pl.ds((row_base + ci * C) // 2, C // 2)]

        def wait_out(ci, s):
            pltpu.make_async_copy(sc_v.at[s], out_slice(ci), osem[s]).wait()

        fire_gather(0, 0)
        fire_gather(1, 1)

        def group_body(g, carry):
            for b in range(2):
                ci = g * 2 + b
                wait_gather(b)

                @pl.when(ci >= 2)
                def _():
                    wait_out(ci - 2, b)

                def scale_pair(q, carry2):
                    for half in range(2):
                        for d in range(D_MODEL // 16):
                            src = ga_v[b, 2 * q + half, pl.ds(d * 16, 16)]
                            sc_v[b, q, pl.ds(half * D_MODEL + d * 16, 16)] = (
                                src * SCALE)
                    return carry2

                lax.fori_loop(0, C // 2, scale_pair, 0, unroll=4)

                pltpu.async_copy(sc_v.at[b], out_slice(ci), osem[b])

                @pl.when(ci + 2 < n_chunks)
                def _():
                    fire_gather(ci + 2, b)
            return carry

        lax.fori_loop(0, n_groups, group_body, 0)
        wait_out(n_chunks - 2, 0)
        wait_out(n_chunks - 1, 1)

    return emb_kernel


def kernel(x, pretrained_vector):
    B = x.shape[0] * x.shape[1]
    idx = x.reshape(B).astype(jnp.int32)
    out = _build(B)(idx, pretrained_vector)
    return out.reshape(x.shape[0], x.shape[1], D_MODEL)


# speedup vs baseline: 1.0504x; 1.0504x over previous
import functools
import math

import jax
import jax.numpy as jnp
from jax import lax
from jax.experimental import pallas as pl
from jax.experimental.pallas import tpu as pltpu
from jax.experimental.pallas import tpu_sc as plsc

D_MODEL = 64
SCALE = math.sqrt(D_MODEL)

G = 128
K = 2
C = G * K


@functools.lru_cache(maxsize=None)
def _build(B):
    info = plsc.get_sparse_core_info()
    NW = info.num_cores * info.num_subcores
    assert B % (NW * C * 2) == 0
    b_per_w = B // NW
    n_chunks = b_per_w // C
    n_groups = n_chunks // 2

    mesh = plsc.VectorSubcoreMesh(core_axis_name="c", subcore_axis_name="s")

    @functools.partial(
        pl.kernel,
        mesh=mesh,
        compiler_params=pltpu.CompilerParams(use_tc_tiling_on_sc=False),
        out_type=jax.ShapeDtypeStruct((B, 2 * D_MODEL), jnp.float32),
        scratch_types=[
            pltpu.VMEM((2, C), jnp.int32),
            pltpu.VMEM((2, C), jnp.int32),
            pltpu.VMEM((2, C, 2 * D_MODEL), jnp.float32),
            pltpu.SemaphoreType.DMA,
            pltpu.SemaphoreType.DMA,
            pltpu.SemaphoreType.DMA,
            pltpu.SemaphoreType.DMA,
        ],
    )
    def emb_kernel(idx_hbm, tbl_hbm, out_hbm, idx_v, jdx_v, ga_v, *sems):
        gsem = sems[:2]
        osem = sems[2:]
        cid = lax.axis_index("c")
        sid = lax.axis_index("s")
        wid = sid * info.num_cores + cid
        row_base = wid * b_per_w

        def stage_and_fire(ci, s):
            src = idx_hbm.at[pl.ds(row_base + ci * C, C)]
            pltpu.sync_copy(src, idx_v.at[s])
            for v in range(C // 16):
                sl = pl.ds(v * 16, 16)
                jdx_v[s, sl] = idx_v[s, sl] >> 1
            for j in range(K):
                pltpu.async_copy(
                    tbl_hbm.at[jdx_v.at[s, pl.ds(j * G, G)]],
                    ga_v.at[s, pl.ds(j * G, G)],
                    gsem[s],
                )

        def wait_gather(s):
            pltpu.make_async_copy(
                tbl_hbm.at[pl.ds(0, C)], ga_v.at[s], gsem[s]).wait()

        def out_slice(ci):
            return out_hbm.at[pl.ds(row_base + ci * C, C)]

        def wait_out(ci, s):
            pltpu.make_async_copy(ga_v.at[s], out_slice(ci), osem[s]).wait()

        stage_and_fire(0, 0)
        stage_and_fire(1, 1)

        def group_body(g, carry):
            for b in range(2):
                ci = g * 2 + b
                wait_gather(b)

                @pl.when(ci >= 2)
                def _():
                    wait_out(ci - 2, b)

                def scale_group(grp, carry2):
                    iv = idx_v[b, pl.ds(grp * 16, 16)]
                    for k in range(16):
                        r = grp * 16 + k
                        off = (iv[k] & 1) * D_MODEL
                        for d in range(D_MODEL // 16):
                            ga_v[b, r, pl.ds(d * 16, 16)] = (
                                ga_v[b, r, pl.ds(off + d * 16, 16)] * SCALE)
                    return carry2

                lax.fori_loop(0, C // 16, scale_group, 0)

                pltpu.async_copy(ga_v.at[b], out_slice(ci), osem[b])

                @pl.when(ci + 2 < n_chunks)
                def _():
                    stage_and_fire(ci + 2, b)
            return carry

        lax.fori_loop(0, n_groups, group_body, 0)
        wait_out(n_chunks - 2, 0)
        wait_out(n_chunks - 1, 1)

    return emb_kernel


def kernel(x, pretrained_vector):
    B = x.shape[0] * x.shape[1]
    idx = x.reshape(B).astype(jnp.int32)
    tbl = pretrained_vector.reshape(-1, 2 * D_MODEL)
    out = _build(B)(idx, tbl)
    out3 = out.reshape(x.shape[0], x.shape[1], 2 * D_MODEL)
    return out3[:, :, :D_MODEL]
